# baseline (speedup 1.0000x reference)
import functools

import jax
import jax.numpy as jnp
from jax.experimental import pallas as pl
from jax.experimental.pallas import tpu as pltpu


def _lrelu(v):
    return jax.nn.leaky_relu(v, 0.01)


def _bn(v, eps=1e-5):
    return (v - v.mean(0)) / jnp.sqrt(v.var(0) + eps)


def _seg_softmax(s, seg, n):
    m = jax.ops.segment_max(s, seg, n)
    ex = jnp.exp(s - m[seg])
    den = jax.ops.segment_sum(ex, seg, n)
    return ex / den[seg]


def _match(e, src, dst, mask, n):
    E = e.shape[0]
    idx = jnp.arange(E, dtype=jnp.int32)
    neg = jnp.float32(-jnp.inf)
    big = jnp.int32(E)

    def cond(c):
        return c[2]

    def body(c):
        rem, taken, _ = c
        act = mask & rem[src] & rem[dst]
        ev = jnp.where(act, e, neg)
        best = jnp.maximum(jax.ops.segment_max(ev, src, n),
                           jax.ops.segment_max(ev, dst, n))
        mi = jnp.minimum(
            jax.ops.segment_min(jnp.where(act & (ev == best[src]), idx, big), src, n),
            jax.ops.segment_min(jnp.where(act & (ev == best[dst]), idx, big), dst, n))
        win = act & (idx == mi[src]) & (idx == mi[dst])
        rem = rem.at[jnp.where(win, src, n)].set(False, mode='drop')
        rem = rem.at[jnp.where(win, dst, n)].set(False, mode='drop')
        taken = taken | win
        cont = (mask & rem[src] & rem[dst]).any()
        return rem, taken, cont

    rem0 = jnp.ones((n,), bool)
    taken0 = jnp.zeros((E,), bool)
    rem, taken, _ = jax.lax.while_loop(cond, body, (rem0, taken0, mask.any()))

    ordk = jnp.argsort(jnp.where(taken, -e, jnp.float32(jnp.inf)))
    rnk = jnp.zeros((E,), jnp.int32).at[ordk].set(
        jnp.arange(E, dtype=jnp.int32))
    i_total = taken.sum(dtype=jnp.int32)
    cluster = jnp.full((n,), -1, dtype=jnp.int32)
    cluster = cluster.at[jnp.where(taken, src, n)].set(
        jnp.where(taken, rnk, 0), mode='drop')
    cluster = cluster.at[jnp.where(taken, dst, n)].set(
        jnp.where(taken, rnk, 0), mode='drop')
    rank = jnp.cumsum(rem.astype(jnp.int32)) - 1
    cluster = jnp.where(rem, i_total + rank, cluster)
    nes = jnp.ones((n,), jnp.float32)
    nes = nes.at[jnp.where(taken, rnk, n)].set(
        jnp.where(taken, e, 1.0), mode='drop')
    return cluster, nes


def _pool(h, ei, mask, lin, n):
    raw = (jnp.concatenate([h[ei[0]], h[ei[1]]], axis=-1) @ lin[0].T
           + lin[1]).reshape(-1)
    seg = jnp.where(mask, ei[1], n)
    e = _seg_softmax(raw, seg, n) + 0.5
    e = jnp.where(mask, e, -jnp.inf)
    cluster, nes = _match(e, ei[0], ei[1], mask, n)
    new_x = jax.ops.segment_sum(h, cluster, n) * nes[:, None]
    return new_x, nes, cluster


def _out_mlp_body(r_ref, w1_ref, b1_ref, w2_ref, b2_ref, w3_ref, b3_ref,
                  o_ref):
    v = r_ref[...]
    v = _lrelu(jnp.dot(v, w1_ref[...], preferred_element_type=jnp.float32)
               + b1_ref[...])
    v = _lrelu(jnp.dot(v, w2_ref[...], preferred_element_type=jnp.float32)
               + b2_ref[...])
    v = jnp.dot(v, w3_ref[...], preferred_element_type=jnp.float32) + b3_ref[...]
    col = jax.lax.broadcasted_iota(jnp.int32, v.shape, 1)
    o_ref[...] = jnp.where(col == 0, jax.nn.sigmoid(v), v)


def _out_mlp(r, P):
    n, hd = r.shape
    h1 = P['out1'][0].shape[0]
    w1 = P['out1'][0].T
    b1 = P['out1'][1][None, :]
    w2 = P['out2'][0].T
    b2 = P['out2'][1][None, :]
    w3 = jnp.zeros((h1, 128), jnp.float32).at[:, :4].set(P['out3'][0].T)
    b3 = jnp.zeros((1, 128), jnp.float32).at[:, :4].set(P['out3'][1][None, :])
    blk = 1000 if n % 1000 == 0 else n
    grid = (n // blk,)
    out = pl.pallas_call(
        _out_mlp_body,
        grid=grid,
        in_specs=[
            pl.BlockSpec((blk, hd), lambda i: (i, 0)),
            pl.BlockSpec((hd, h1), lambda i: (0, 0)),
            pl.BlockSpec((1, h1), lambda i: (0, 0)),
            pl.BlockSpec((h1, h1), lambda i: (0, 0)),
            pl.BlockSpec((1, h1), lambda i: (0, 0)),
            pl.BlockSpec((h1, 128), lambda i: (0, 0)),
            pl.BlockSpec((1, 128), lambda i: (0, 0)),
        ],
        out_specs=pl.BlockSpec((blk, 128), lambda i: (i, 0)),
        out_shape=jax.ShapeDtypeStruct((n, 128), jnp.float32),
    )(r, w1, b1, w2, b2, w3, b3)
    return out[:, :4]


def kernel(x, edge_attr, params, edge_index, batch):
    P = params
    src = edge_index[0]
    dst = edge_index[1]
    n = x.shape[0]

    Hh = _lrelu(x @ P['in1'][0].T + P['in1'][1])
    Hh = _lrelu(Hh @ P['in2'][0].T + P['in2'][1])
    h = jnp.concatenate([Hh, x], axis=-1)
    h = _bn(h)

    xi = h[dst]
    xj = h[src]
    m = jnp.concatenate([xi, xj - xi, edge_attr], axis=-1)
    m = _lrelu(m @ P['phi1'][0].T + P['phi1'][1])
    m = _lrelu(m @ P['phi2'][0].T + P['phi2'][1])
    m = _lrelu(m @ P['phi3'][0].T + P['phi3'][1])
    ssum = jax.ops.segment_sum(m, dst, n)
    cnt = jax.ops.segment_sum(jnp.ones((m.shape[0],), m.dtype), dst, n)
    agg = ssum / jnp.maximum(cnt, 1.0)[:, None]
    g = _lrelu(agg @ P['gam1'][0].T + P['gam1'][1])
    g = _lrelu(g @ P['gam2'][0].T + P['gam2'][1])
    g = _lrelu(g @ P['gam3'][0].T + P['gam3'][1])
    h = jnp.concatenate([g, x], axis=-1)
    h = _bn(h)

    mask0 = jnp.ones((edge_index.shape[1],), dtype=bool)
    r1, nes1, cluster1 = _pool(h, edge_index, mask0, P['pool1'], n)

    c0 = cluster1[src]
    c1 = cluster1[dst]
    keys = jnp.where(mask0, c0 * n + c1, n * n)
    sk = jnp.sort(keys)
    first = jnp.concatenate([jnp.ones((1,), dtype=bool), sk[1:] != sk[:-1]])
    new_mask = first & (sk < n * n)
    sk = jnp.where(new_mask, sk, 0)
    new_ei = jnp.stack([sk // n, sk % n]).astype(jnp.int32)

    r2, nes2, cluster2 = _pool(r1, new_ei, new_mask, P['pool2'], n)

    r = (r2 / nes2[:, None])[cluster2]
    r = (r / nes1[:, None])[cluster1]
    return _out_mlp(r, P)


# speedup vs baseline: 10.4618x; 10.4618x over previous
import functools

import jax
import jax.numpy as jnp
from jax.experimental import pallas as pl
from jax.experimental.pallas import tpu as pltpu


def _lrelu(v):
    return jax.nn.leaky_relu(v, 0.01)


def _bn(v, eps=1e-5):
    return (v - v.mean(0)) / jnp.sqrt(v.var(0) + eps)


def _seg_softmax(s, seg, n):
    m = jax.ops.segment_max(s, seg, n)
    ex = jnp.exp(s - m[seg])
    den = jax.ops.segment_sum(ex, seg, n)
    return ex / den[seg]


def _greedy_body(ss_ref, ds_ref, es_ref, ms_ref, cluster_ref, nes_ref,
                 rem_ref, i_ref):
    step = pl.program_id(0)
    nb = pl.num_programs(0)
    be = ss_ref.shape[0]
    n = cluster_ref.shape[0]

    @pl.when(step == 0)
    def _init():
        def ib(v, c):
            rem_ref[v] = jnp.int32(1)
            nes_ref[v] = jnp.float32(1.0)
            return c
        jax.lax.fori_loop(0, n, ib, jnp.int32(0))
        i_ref[0] = jnp.int32(0)

    def eb(k, i):
        s = ss_ref[k]
        t = ds_ref[k]
        take = (ms_ref[k] == 1) & (rem_ref[s] == 1) & (rem_ref[t] == 1)

        @pl.when(take)
        def _():
            rem_ref[s] = jnp.int32(0)
            rem_ref[t] = jnp.int32(0)
            cluster_ref[s] = i
            cluster_ref[t] = i
            nes_ref[i] = es_ref[k]

        return i + take.astype(jnp.int32)

    i_ref[0] = jax.lax.fori_loop(0, be, eb, i_ref[0])

    @pl.when(step == nb - 1)
    def _final():
        itot = i_ref[0]

        def vb(v, rank):
            r = rem_ref[v] == 1

            @pl.when(r)
            def _():
                cluster_ref[v] = itot + rank

            return rank + r.astype(jnp.int32)

        jax.lax.fori_loop(0, n, vb, jnp.int32(0))


def _match(e, src, dst, mask, n):
    E = e.shape[0]
    _, ss, ds, ms, es = jax.lax.sort(
        (-e, src, dst, mask.astype(jnp.int32), e), num_keys=1)
    be = 2048
    ep = ((E + be - 1) // be) * be
    if ep != E:
        pad = ep - E
        ss = jnp.pad(ss, (0, pad))
        ds = jnp.pad(ds, (0, pad))
        ms = jnp.pad(ms, (0, pad))
        es = jnp.pad(es, (0, pad))
    E = ep
    cluster, nes = pl.pallas_call(
        _greedy_body,
        grid=(E // be,),
        in_specs=[
            pl.BlockSpec((be,), lambda i: (i,), memory_space=pltpu.SMEM),
            pl.BlockSpec((be,), lambda i: (i,), memory_space=pltpu.SMEM),
            pl.BlockSpec((be,), lambda i: (i,), memory_space=pltpu.SMEM),
            pl.BlockSpec((be,), lambda i: (i,), memory_space=pltpu.SMEM),
        ],
        out_specs=[
            pl.BlockSpec((n,), lambda i: (0,), memory_space=pltpu.SMEM),
            pl.BlockSpec((n,), lambda i: (0,), memory_space=pltpu.SMEM),
        ],
        out_shape=[
            jax.ShapeDtypeStruct((n,), jnp.int32),
            jax.ShapeDtypeStruct((n,), jnp.float32),
        ],
        scratch_shapes=[
            pltpu.SMEM((n,), jnp.int32),
            pltpu.SMEM((1,), jnp.int32),
        ],
    )(ss, ds, es, ms)
    return cluster, nes


def _pool(h, ei, mask, lin, n):
    raw = (jnp.concatenate([h[ei[0]], h[ei[1]]], axis=-1) @ lin[0].T
           + lin[1]).reshape(-1)
    seg = jnp.where(mask, ei[1], n)
    e = _seg_softmax(raw, seg, n) + 0.5
    e = jnp.where(mask, e, -jnp.inf)
    cluster, nes = _match(e, ei[0], ei[1], mask, n)
    new_x = jax.ops.segment_sum(h, cluster, n) * nes[:, None]
    return new_x, nes, cluster


def _out_mlp_body(r_ref, w1_ref, b1_ref, w2_ref, b2_ref, w3_ref, b3_ref,
                  o_ref):
    v = r_ref[...]
    v = _lrelu(jnp.dot(v, w1_ref[...], preferred_element_type=jnp.float32)
               + b1_ref[...])
    v = _lrelu(jnp.dot(v, w2_ref[...], preferred_element_type=jnp.float32)
               + b2_ref[...])
    v = jnp.dot(v, w3_ref[...], preferred_element_type=jnp.float32) + b3_ref[...]
    col = jax.lax.broadcasted_iota(jnp.int32, v.shape, 1)
    o_ref[...] = jnp.where(col == 0, jax.nn.sigmoid(v), v)


def _out_mlp(r, P):
    n, hd = r.shape
    h1 = P['out1'][0].shape[0]
    w1 = P['out1'][0].T
    b1 = P['out1'][1][None, :]
    w2 = P['out2'][0].T
    b2 = P['out2'][1][None, :]
    w3 = jnp.zeros((h1, 128), jnp.float32).at[:, :4].set(P['out3'][0].T)
    b3 = jnp.zeros((1, 128), jnp.float32).at[:, :4].set(P['out3'][1][None, :])
    blk = 1000 if n % 1000 == 0 else n
    grid = (n // blk,)
    out = pl.pallas_call(
        _out_mlp_body,
        grid=grid,
        in_specs=[
            pl.BlockSpec((blk, hd), lambda i: (i, 0)),
            pl.BlockSpec((hd, h1), lambda i: (0, 0)),
            pl.BlockSpec((1, h1), lambda i: (0, 0)),
            pl.BlockSpec((h1, h1), lambda i: (0, 0)),
            pl.BlockSpec((1, h1), lambda i: (0, 0)),
            pl.BlockSpec((h1, 128), lambda i: (0, 0)),
            pl.BlockSpec((1, 128), lambda i: (0, 0)),
        ],
        out_specs=pl.BlockSpec((blk, 128), lambda i: (i, 0)),
        out_shape=jax.ShapeDtypeStruct((n, 128), jnp.float32),
    )(r, w1, b1, w2, b2, w3, b3)
    return out[:, :4]


def kernel(x, edge_attr, params, edge_index, batch):
    P = params
    src = edge_index[0]
    dst = edge_index[1]
    n = x.shape[0]

    Hh = _lrelu(x @ P['in1'][0].T + P['in1'][1])
    Hh = _lrelu(Hh @ P['in2'][0].T + P['in2'][1])
    h = jnp.concatenate([Hh, x], axis=-1)
    h = _bn(h)

    xi = h[dst]
    xj = h[src]
    m = jnp.concatenate([xi, xj - xi, edge_attr], axis=-1)
    m = _lrelu(m @ P['phi1'][0].T + P['phi1'][1])
    m = _lrelu(m @ P['phi2'][0].T + P['phi2'][1])
    m = _lrelu(m @ P['phi3'][0].T + P['phi3'][1])
    ssum = jax.ops.segment_sum(m, dst, n)
    cnt = jax.ops.segment_sum(jnp.ones((m.shape[0],), m.dtype), dst, n)
    agg = ssum / jnp.maximum(cnt, 1.0)[:, None]
    g = _lrelu(agg @ P['gam1'][0].T + P['gam1'][1])
    g = _lrelu(g @ P['gam2'][0].T + P['gam2'][1])
    g = _lrelu(g @ P['gam3'][0].T + P['gam3'][1])
    h = jnp.concatenate([g, x], axis=-1)
    h = _bn(h)

    mask0 = jnp.ones((edge_index.shape[1],), dtype=bool)
    r1, nes1, cluster1 = _pool(h, edge_index, mask0, P['pool1'], n)

    c0 = cluster1[src]
    c1 = cluster1[dst]
    keys = jnp.where(mask0, c0 * n + c1, n * n)
    sk = jnp.sort(keys)
    first = jnp.concatenate([jnp.ones((1,), dtype=bool), sk[1:] != sk[:-1]])
    new_mask = first & (sk < n * n)
    sk = jnp.where(new_mask, sk, 0)
    new_ei = jnp.stack([sk // n, sk % n]).astype(jnp.int32)

    r2, nes2, cluster2 = _pool(r1, new_ei, new_mask, P['pool2'], n)

    r = (r2 / nes2[:, None])[cluster2]
    r = (r / nes1[:, None])[cluster1]
    return _out_mlp(r, P)
